# baseline (speedup 1.0000x reference)
import jax
import jax.numpy as jnp
from jax import lax
from jax.experimental import pallas as pl
from jax.experimental.pallas import tpu as pltpu
from jax.experimental.pallas import tpu_sc as plsc

M = 32
NUM_CODES = 256
EMB_DIM = 16
BATCH = 16384
NSPLIT = 2
B_FLAT_H = BATCH * M // NSPLIT
NC, NS = 2, 16
NW = NC * NS
ROWS_PER_W = B_FLAT_H // NW
G = 128
GPC = 16
CHUNK_ROWS = GPC * G
NCHUNK = ROWS_PER_W // CHUNK_ROWS
L = 16


def _sc_body(codes_hbm, table_hbm, out_hbm,
             codes_v, idx_v, rows_v, shared_tab,
             sem_tab, sem_codes, sem_g, sem_o0, sem_o1):
    sid = lax.axis_index("s")
    wid = sid * NC + lax.axis_index("c")
    base = wid * ROWS_PER_W

    @pl.when(sid == 0)
    def _():
        tcopies = [
            pltpu.make_async_copy(
                table_hbm.at[i], shared_tab.at[pl.ds(i * NUM_CODES,
                                                     NUM_CODES)], sem_tab)
            for i in range(M)
        ]
        for tc in tcopies:
            tc.start()
        for tc in tcopies:
            tc.wait()

    pltpu.async_copy(codes_hbm.at[pl.ds(base, ROWS_PER_W)], codes_v,
                     sem_codes).wait()
    plsc.subcore_barrier()

    tab_flat = shared_tab

    off_e = lax.iota(jnp.int32, L) * NUM_CODES
    off_o = off_e + L * NUM_CODES
    out_sems = (sem_o0, sem_o1)

    def do_chunk(c, slot):
        @pl.when(c >= 2)
        def _():
            pltpu.make_async_copy(
                rows_v.at[slot], out_hbm.at[pl.ds(base, CHUNK_ROWS)],
                out_sems[slot]).wait()

        p0 = c * CHUNK_ROWS
        for g in range(GPC):
            for k in range(G // L):
                off = off_e if k % 2 == 0 else off_o
                idx_v[slot, g, pl.ds(k * L, L)] = (
                    codes_v[pl.ds(p0 + g * G + k * L, L)] + off)

        copies = [
            pltpu.async_copy(tab_flat.at[idx_v.at[slot, g]],
                             rows_v.at[slot, pl.ds(g * G, G)], sem_g)
            for g in range(GPC)
        ]
        for cp in copies:
            cp.wait()

        pltpu.async_copy(rows_v.at[slot],
                         out_hbm.at[pl.ds(base + p0, CHUNK_ROWS)],
                         out_sems[slot])

    def pair(i, carry):
        do_chunk(2 * i, 0)
        do_chunk(2 * i + 1, 1)
        return carry

    lax.fori_loop(0, NCHUNK // 2, pair, None)

    for slot in range(2):
        pltpu.make_async_copy(rows_v.at[slot],
                              out_hbm.at[pl.ds(base, CHUNK_ROWS)],
                              out_sems[slot]).wait()


_pq_gather = pl.kernel(
    _sc_body,
    out_type=jax.ShapeDtypeStruct((B_FLAT_H, EMB_DIM), jnp.float32),
    mesh=plsc.VectorSubcoreMesh(core_axis_name="c", subcore_axis_name="s"),
    compiler_params=pltpu.CompilerParams(use_tc_tiling_on_sc=False),
    scratch_types=[
        pltpu.VMEM((ROWS_PER_W,), jnp.int32),
        pltpu.VMEM((2, GPC, G), jnp.int32),
        pltpu.VMEM((2, CHUNK_ROWS, EMB_DIM), jnp.float32),
        pltpu.VMEM_SHARED((M * NUM_CODES, EMB_DIM), jnp.float32),
        pltpu.SemaphoreType.DMA,
        pltpu.SemaphoreType.DMA,
        pltpu.SemaphoreType.DMA,
        pltpu.SemaphoreType.DMA,
        pltpu.SemaphoreType.DMA,
    ],
)


def kernel(pq_codes, tables):
    codes_flat = pq_codes.reshape(-1).astype(jnp.int32)
    halves = []
    for h in range(NSPLIT):
        out_h = _pq_gather(
            lax.slice_in_dim(codes_flat, h * B_FLAT_H, (h + 1) * B_FLAT_H),
            tables)
        halves.append(out_h.reshape(BATCH // NSPLIT, M * EMB_DIM))
    return jnp.concatenate(halves, axis=0)


# speedup vs baseline: 1.2269x; 1.2269x over previous
import jax
import jax.numpy as jnp
from jax import lax
from jax.experimental import pallas as pl
from jax.experimental.pallas import tpu as pltpu
from jax.experimental.pallas import tpu_sc as plsc

M = 32
NUM_CODES = 256
EMB_DIM = 16
BATCH = 16384
B_FLAT = BATCH * M
NC, NS = 2, 16
NW = NC * NS
BATCH_PER_W = BATCH // NW
ROWS_PER_W = B_FLAT // NW
G = 128
GPC = 16
CHUNK_ROWS = GPC * G
NCHUNK = ROWS_PER_W // CHUNK_ROWS
L = 16


def _sc_body(codes_hbm, table_hbm, out_hbm,
             codes_v, idx_v, rows_v, shared_tab,
             sem_tab, sem_codes, sem_g, sem_o0, sem_o1):
    sid = lax.axis_index("s")
    wid = sid * NC + lax.axis_index("c")
    base = wid * ROWS_PER_W

    ccopy = pltpu.async_copy(
        codes_hbm.at[pl.ds(wid * BATCH_PER_W, BATCH_PER_W)], codes_v,
        sem_codes)

    @pl.when(sid == 0)
    def _():
        tcopies = [
            pltpu.make_async_copy(
                table_hbm.at[i],
                shared_tab.at[pl.ds(i * NUM_CODES, NUM_CODES)], sem_tab)
            for i in range(M)
        ]
        for tc in tcopies:
            tc.start()
        for tc in tcopies:
            tc.wait()

    ccopy.wait()
    plsc.subcore_barrier()

    off_lo = lax.iota(jnp.int32, L) * NUM_CODES
    off_hi = off_lo + L * NUM_CODES
    out_sems = (sem_o0, sem_o1)

    def do_chunk(c, slot):
        @pl.when(c >= 2)
        def _():
            pltpu.make_async_copy(
                rows_v.at[slot], out_hbm.at[pl.ds(base, CHUNK_ROWS)],
                out_sems[slot]).wait()

        for g in range(GPC):
            for rr in range(G // M):
                row = c * (CHUNK_ROWS // M) + g * (G // M) + rr
                idx_v[slot, g, pl.ds(rr * M, L)] = (
                    codes_v[row, pl.ds(0, L)] + off_lo)
                idx_v[slot, g, pl.ds(rr * M + L, L)] = (
                    codes_v[row, pl.ds(L, L)] + off_hi)

        copies = [
            pltpu.async_copy(shared_tab.at[idx_v.at[slot, g]],
                             rows_v.at[slot, pl.ds(g * G, G)], sem_g)
            for g in range(GPC)
        ]
        for cp in copies:
            cp.wait()

        pltpu.async_copy(rows_v.at[slot],
                         out_hbm.at[pl.ds(base + c * CHUNK_ROWS, CHUNK_ROWS)],
                         out_sems[slot])

    def pair(i, carry):
        do_chunk(2 * i, 0)
        do_chunk(2 * i + 1, 1)
        return carry

    lax.fori_loop(0, NCHUNK // 2, pair, None)

    for slot in range(2):
        pltpu.make_async_copy(rows_v.at[slot],
                              out_hbm.at[pl.ds(base, CHUNK_ROWS)],
                              out_sems[slot]).wait()


_pq_gather = pl.kernel(
    _sc_body,
    out_type=jax.ShapeDtypeStruct((B_FLAT, EMB_DIM), jnp.float32),
    mesh=plsc.VectorSubcoreMesh(core_axis_name="c", subcore_axis_name="s"),
    compiler_params=pltpu.CompilerParams(use_tc_tiling_on_sc=False),
    scratch_types=[
        pltpu.VMEM((BATCH_PER_W, M), jnp.int32),
        pltpu.VMEM((2, GPC, G), jnp.int32),
        pltpu.VMEM((2, CHUNK_ROWS, EMB_DIM), jnp.float32),
        pltpu.VMEM_SHARED((M * NUM_CODES, EMB_DIM), jnp.float32),
        pltpu.SemaphoreType.DMA,
        pltpu.SemaphoreType.DMA,
        pltpu.SemaphoreType.DMA,
        pltpu.SemaphoreType.DMA,
        pltpu.SemaphoreType.DMA,
    ],
)


def kernel(pq_codes, tables):
    out = _pq_gather(pq_codes.astype(jnp.int32), tables)
    return out.reshape(BATCH, M * EMB_DIM)
